# Initial kernel scaffold; baseline (speedup 1.0000x reference)
import jax
import jax.numpy as jnp
from jax.experimental import pallas as pl


def kernel(node_latents, mesh_edge_latents, We1, be1, We2, be2, ge, bbe, Wn1, bn1, Wn2, bn2, gn, bbn, senders, receivers):
    raise NotImplementedError("write your pallas kernel here")


# speedup vs baseline: 3.8605x; 3.8605x over previous
import functools

import jax
import jax.numpy as jnp
from jax import lax
from jax.experimental import pallas as pl
from jax.experimental.pallas import tpu as pltpu
from jax.experimental.pallas import tpu_sc as plsc

LATENT = 128
N_NODES = 10000
N_EDGES = 320000
NC, NS = 2, 16
NW = NC * NS
EPW = N_EDGES // NW
CHUNK = 80
NCHUNK = EPW // CHUNK
N_PAD = 10240
ROWS_PER_TILE = N_PAD // NS

_F32 = jnp.float32
_EPS = 1e-5

_sc_mesh = plsc.VectorSubcoreMesh(core_axis_name="c", subcore_axis_name="s")


def _proj_body(n_ref, wa_ref, wb_ref, ps_ref, pr_ref):
    n = n_ref[...]
    ps_ref[...] = jnp.dot(n, wa_ref[...], preferred_element_type=_F32)
    pr_ref[...] = jnp.dot(n, wb_ref[...], preferred_element_type=_F32)


def _project(nodes, wa, wb):
    B = 2000
    return pl.pallas_call(
        _proj_body,
        grid=(N_NODES // B,),
        in_specs=[
            pl.BlockSpec((B, LATENT), lambda i: (i, 0)),
            pl.BlockSpec((LATENT, LATENT), lambda i: (0, 0)),
            pl.BlockSpec((LATENT, LATENT), lambda i: (0, 0)),
        ],
        out_specs=[pl.BlockSpec((B, LATENT), lambda i: (i, 0))] * 2,
        out_shape=[jax.ShapeDtypeStruct((N_NODES, LATENT), _F32)] * 2,
    )(nodes, wa, wb)


def _edge_body(g1_ref, g2_ref, m_ref, wm_ref, w2_ref, b1_ref, b2_ref, g_ref,
               b_ref, new_ref, out_ref):
    mesh = m_ref[...]
    x = (g1_ref[...] + g2_ref[...] + b1_ref[...]
         + jnp.dot(mesh, wm_ref[...], preferred_element_type=_F32))
    h = jnp.maximum(x, 0.0)
    h = jnp.dot(h, w2_ref[...], preferred_element_type=_F32) + b2_ref[...]
    h = jnp.maximum(h, 0.0)
    mu = jnp.mean(h, axis=-1, keepdims=True)
    var = jnp.mean((h - mu) * (h - mu), axis=-1, keepdims=True)
    ln = (h - mu) * lax.rsqrt(var + _EPS) * g_ref[...] + b_ref[...]
    new_ref[...] = ln
    out_ref[...] = ln + mesh


def _edge_mlp(g1, g2, mesh, wm, w2, b1, b2, g, b):
    E = 3200
    full = lambda i: (0, 0)
    row = lambda i: (i, 0)
    return pl.pallas_call(
        _edge_body,
        grid=(N_EDGES // E,),
        in_specs=[
            pl.BlockSpec((E, LATENT), row),
            pl.BlockSpec((E, LATENT), row),
            pl.BlockSpec((E, LATENT), row),
            pl.BlockSpec((LATENT, LATENT), full),
            pl.BlockSpec((LATENT, LATENT), full),
            pl.BlockSpec((1, LATENT), full),
            pl.BlockSpec((1, LATENT), full),
            pl.BlockSpec((1, LATENT), full),
            pl.BlockSpec((1, LATENT), full),
        ],
        out_specs=[pl.BlockSpec((E, LATENT), row)] * 2,
        out_shape=[jax.ShapeDtypeStruct((N_EDGES, LATENT), _F32)] * 2,
    )(g1, g2, mesh, wm, w2, b1, b2, g, b)


def _node_body(n_ref, agg_ref, wa_ref, wb_ref, w2_ref, b1_ref, b2_ref, g_ref,
               b_ref, out_ref):
    nodes = n_ref[...]
    a = agg_ref[0] + agg_ref[1]
    x = (jnp.dot(nodes, wa_ref[...], preferred_element_type=_F32)
         + jnp.dot(a, wb_ref[...], preferred_element_type=_F32) + b1_ref[...])
    h = jnp.maximum(x, 0.0)
    h = jnp.dot(h, w2_ref[...], preferred_element_type=_F32) + b2_ref[...]
    h = jnp.maximum(h, 0.0)
    mu = jnp.mean(h, axis=-1, keepdims=True)
    var = jnp.mean((h - mu) * (h - mu), axis=-1, keepdims=True)
    ln = (h - mu) * lax.rsqrt(var + _EPS) * g_ref[...] + b_ref[...]
    out_ref[...] = ln + nodes


def _node_mlp(nodes, partials, wa, wb, w2, b1, b2, g, b):
    B = 2000
    full = lambda i: (0, 0)
    return pl.pallas_call(
        _node_body,
        grid=(N_NODES // B,),
        in_specs=[
            pl.BlockSpec((B, LATENT), lambda i: (i, 0)),
            pl.BlockSpec((NC, B, LATENT), lambda i: (0, i, 0)),
            pl.BlockSpec((LATENT, LATENT), full),
            pl.BlockSpec((LATENT, LATENT), full),
            pl.BlockSpec((LATENT, LATENT), full),
            pl.BlockSpec((1, LATENT), full),
            pl.BlockSpec((1, LATENT), full),
            pl.BlockSpec((1, LATENT), full),
            pl.BlockSpec((1, LATENT), full),
        ],
        out_specs=pl.BlockSpec((B, LATENT), lambda i: (i, 0)),
        out_shape=jax.ShapeDtypeStruct((N_NODES, LATENT), _F32),
    )(nodes, partials, wa, wb, w2, b1, b2, g, b)


@functools.partial(
    pl.kernel,
    out_type=[
        jax.ShapeDtypeStruct((N_EDGES, LATENT), _F32),
        jax.ShapeDtypeStruct((N_EDGES, LATENT), _F32),
    ],
    mesh=_sc_mesh,
    scratch_types=[
        pltpu.VMEM((NCHUNK, CHUNK), jnp.int32),
        pltpu.VMEM((NCHUNK, CHUNK), jnp.int32),
        pltpu.VMEM((CHUNK, LATENT), _F32),
        pltpu.VMEM((CHUNK, LATENT), _F32),
        pltpu.SemaphoreType.DMA,
        pltpu.SemaphoreType.DMA,
    ],
)
def _sc_gather(ps_hbm, pr_hbm, sidx_hbm, ridx_hbm, g1_hbm, g2_hbm,
               sidx_v, ridx_v, buf1, buf2, sem1, sem2):
    wid = lax.axis_index("s") * NC + lax.axis_index("c")
    cbase = wid * NCHUNK
    pltpu.sync_copy(sidx_hbm.at[wid], sidx_v)
    pltpu.sync_copy(ridx_hbm.at[wid], ridx_v)

    @pl.loop(0, NCHUNK)
    def _chunk(j):
        d1 = pltpu.async_copy(ps_hbm.at[sidx_v.at[j]], buf1, sem1)
        d2 = pltpu.async_copy(pr_hbm.at[ridx_v.at[j]], buf2, sem2)
        d1.wait()
        d2.wait()
        row0 = (cbase + j) * CHUNK
        o1 = pltpu.async_copy(buf1, g1_hbm.at[pl.ds(row0, CHUNK)], sem1)
        o2 = pltpu.async_copy(buf2, g2_hbm.at[pl.ds(row0, CHUNK)], sem2)
        o1.wait()
        o2.wait()


@functools.partial(
    pl.kernel,
    out_type=jax.ShapeDtypeStruct((NC, N_PAD, LATENT), _F32),
    mesh=_sc_mesh,
    scratch_types=[
        pltpu.VMEM((NCHUNK, CHUNK), jnp.int32),
        pltpu.VMEM((CHUNK, LATENT), _F32),
        pltpu.VMEM_SHARED((N_PAD, LATENT), _F32),
        pltpu.SemaphoreType.DMA,
    ],
)
def _sc_scatter(e_hbm, ridx_hbm, zero_hbm, out_hbm, ridx_v, ebuf, acc, sem):
    cid = lax.axis_index("c")
    sid = lax.axis_index("s")
    wid = sid * NC + cid
    my_rows = pl.ds(sid * ROWS_PER_TILE, ROWS_PER_TILE)
    pltpu.sync_copy(zero_hbm, acc.at[my_rows])
    plsc.subcore_barrier()
    cbase = wid * NCHUNK
    pltpu.sync_copy(ridx_hbm.at[wid], ridx_v)

    @pl.loop(0, NCHUNK)
    def _chunk(j):
        pltpu.async_copy(
            e_hbm.at[pl.ds((cbase + j) * CHUNK, CHUNK)], ebuf, sem).wait()
        pltpu.sync_copy(ebuf, acc.at[ridx_v.at[j]], add=True)

    plsc.subcore_barrier()
    pltpu.sync_copy(acc.at[my_rows], out_hbm.at[cid, my_rows])


def kernel(node_latents, mesh_edge_latents, We1, be1, We2, be2, ge, bbe,
           Wn1, bn1, Wn2, bn2, gn, bbn, senders, receivers):
    wa = We1[:LATENT]
    wb = We1[LATENT:2 * LATENT]
    wm = We1[2 * LATENT:]
    ps, pr = _project(node_latents, wa, wb)

    s3 = senders.reshape(NW, NCHUNK, CHUNK)
    r3 = receivers.reshape(NW, NCHUNK, CHUNK)
    g1, g2 = _sc_gather(ps, pr, s3, r3)

    new_e, out_e = _edge_mlp(
        g1, g2, mesh_edge_latents, wm, We2,
        be1.reshape(1, LATENT), be2.reshape(1, LATENT),
        ge.reshape(1, LATENT), bbe.reshape(1, LATENT))

    zeros = jnp.zeros((ROWS_PER_TILE, LATENT), _F32)
    partials = _sc_scatter(new_e, r3, zeros)

    new_n = _node_mlp(
        node_latents, partials[:, :N_NODES], Wn1[:LATENT], Wn1[LATENT:], Wn2,
        bn1.reshape(1, LATENT), bn2.reshape(1, LATENT),
        gn.reshape(1, LATENT), bbn.reshape(1, LATENT))

    return (new_n, out_e)
